# baseline (speedup 1.0000x reference)
import functools

import jax
import jax.numpy as jnp
from jax import lax
from jax.experimental import pallas as pl
from jax.experimental.pallas import tpu as pltpu
from jax.experimental.pallas import tpu_sc as plsc

N = 10000
D = 128
E = 320000
NC = 2
NS = 16
NW = NC * NS
EPW = E // NW
CHUNK = 80
RING = 3
NCHUNK = 129
NROUND = NCHUNK // RING
EPWP = NCHUNK * CHUNK
NP = 10240
RPS = NP // NS
EPWPX = EPWP + RING * CHUNK
BLK = 1000


def _zero_table(zrow_hbm, rows_v, agg_sh, r0):
  pltpu.sync_copy(zrow_hbm, rows_v)
  for t in range(RPS // CHUNK):
    pltpu.sync_copy(rows_v, agg_sh.at[pl.ds(r0 + t * CHUNK, CHUNK)])


def _copy_out(agg_sh, rows_v, agg_out, cid, r0):
  for t in range(RPS // CHUNK):
    pltpu.sync_copy(agg_sh.at[pl.ds(r0 + t * CHUNK, CHUNK)], rows_v)
    pltpu.sync_copy(rows_v, agg_out.at[pl.ds(cid * NP + r0 + t * CHUNK, CHUNK)])


_seg_mesh = plsc.VectorSubcoreMesh(core_axis_name="c", subcore_axis_name="s")


@functools.partial(
    pl.kernel, mesh=_seg_mesh,
    out_type=jax.ShapeDtypeStruct((NC * NP, D), jnp.float32),
    scratch_types=(
        [pltpu.VMEM((CHUNK,), jnp.int32)] * RING
        + [pltpu.VMEM((CHUNK,), jnp.int32)] * RING
        + [pltpu.VMEM((CHUNK, D), jnp.float32)] * RING
        + [pltpu.VMEM_SHARED((NP, D), jnp.float32)]
        + [pltpu.SemaphoreType.DMA] * RING
        + [pltpu.SemaphoreType.DMA] * RING
    ))
def _seg(z_hbm, src_hbm, dst_hbm, zrow_hbm, agg_out, *refs):
  srcs = refs[0:RING]
  dsts = refs[RING:2 * RING]
  rows = refs[2 * RING:3 * RING]
  agg_sh = refs[3 * RING]
  sem_g = refs[3 * RING + 1:3 * RING + 1 + RING]
  sem_s = refs[3 * RING + 1 + RING:3 * RING + 1 + 2 * RING]
  cid = lax.axis_index("c")
  sid = lax.axis_index("s")
  wid = sid * NC + cid
  r0 = sid * RPS
  _zero_table(zrow_hbm, rows[0], agg_sh, r0)
  plsc.subcore_barrier()

  ebase = wid * EPWPX

  def _stage_and_gather(c, b):
    off = ebase + c * CHUNK
    pltpu.sync_copy(src_hbm.at[pl.ds(off, CHUNK)], srcs[b])
    pltpu.sync_copy(dst_hbm.at[pl.ds(off, CHUNK)], dsts[b])
    pltpu.async_copy(z_hbm.at[srcs[b]], rows[b], sem_g[b])

  for b in range(RING):
    _stage_and_gather(b, b)

  def round_fn(g, carry):
    c0 = g * RING
    for b in range(RING):
      pltpu.make_async_copy(z_hbm.at[srcs[b]], rows[b], sem_g[b]).wait()
      pltpu.async_copy(rows[b], agg_sh.at[dsts[b]], sem_s[b], add=True)
    for b in range(RING):
      pltpu.make_async_copy(rows[b], agg_sh.at[dsts[b]], sem_s[b]).wait()
      _stage_and_gather(c0 + RING + b, b)
    return carry

  lax.fori_loop(0, NROUND, round_fn, 0)
  for b in range(RING):
    pltpu.make_async_copy(z_hbm.at[srcs[b]], rows[b], sem_g[b]).wait()
  plsc.subcore_barrier()
  _copy_out(agg_sh, rows[0], agg_out, cid, r0)


@functools.partial(
    pl.kernel, mesh=_seg_mesh,
    out_type=jax.ShapeDtypeStruct((NC * NP, D), jnp.float32),
    scratch_types=[
        pltpu.VMEM((CHUNK,), jnp.int32),
        pltpu.VMEM((CHUNK,), jnp.int32),
        pltpu.VMEM((CHUNK, D), jnp.float32),
        pltpu.VMEM_SHARED((NP, D), jnp.float32),
        pltpu.SemaphoreType.DMA,
    ])
def _cntk(ones_hbm, src_hbm, dst_hbm, zrow_hbm, cnt_out,
          dst_a, dst_b, rows_v, cnt_sh, sem):
  del src_hbm
  cid = lax.axis_index("c")
  sid = lax.axis_index("s")
  wid = sid * NC + cid
  r0 = sid * RPS
  _zero_table(zrow_hbm, rows_v, cnt_sh, r0)
  pltpu.sync_copy(ones_hbm, rows_v)
  plsc.subcore_barrier()

  ebase = wid * EPWPX
  pltpu.sync_copy(dst_hbm.at[pl.ds(ebase, CHUNK)], dst_a)

  def step(j, carry):
    c1 = 2 * j + 1
    c2 = 2 * j + 2
    pltpu.sync_copy(dst_hbm.at[pl.ds(ebase + c1 * CHUNK, CHUNK)], dst_b)
    pltpu.sync_copy(rows_v, cnt_sh.at[dst_a], add=True)
    pltpu.sync_copy(dst_hbm.at[pl.ds(ebase + c2 * CHUNK, CHUNK)], dst_a)
    pltpu.sync_copy(rows_v, cnt_sh.at[dst_b], add=True)
    return carry

  lax.fori_loop(0, NCHUNK // 2, step, 0)
  plsc.subcore_barrier()
  _copy_out(cnt_sh, rows_v, cnt_out, cid, r0)


def _lin2_body(x_ref, wl_ref, wr_ref, b_ref, z_ref, r_ref):
  x = x_ref[...]
  dn = (((1,), (1,)), ((), ()))
  z_ref[...] = lax.dot_general(x, wl_ref[...], dn,
                               preferred_element_type=jnp.float32)
  r_ref[...] = lax.dot_general(x, wr_ref[...], dn,
                               preferred_element_type=jnp.float32) + b_ref[...]


_lin2 = pl.pallas_call(
    _lin2_body,
    grid=(N // BLK,),
    in_specs=[
        pl.BlockSpec((BLK, D), lambda i: (i, 0)),
        pl.BlockSpec((D, D), lambda i: (0, 0)),
        pl.BlockSpec((D, D), lambda i: (0, 0)),
        pl.BlockSpec((1, D), lambda i: (0, 0)),
    ],
    out_specs=[pl.BlockSpec((BLK, D), lambda i: (i, 0)),
               pl.BlockSpec((BLK, D), lambda i: (i, 0))],
    out_shape=[jax.ShapeDtypeStruct((N, D), jnp.float32)] * 2,
)


def _mean_of(aggp, cntp, res):
  agg = aggp[0] + aggp[1]
  cnt = cntp[0, :, 0:1] + cntp[1, :, 0:1]
  return agg / jnp.clip(cnt, 1.0, None) + res


def _mid_body(aggp_ref, cntp_ref, r1_ref, wl_ref, wr_ref, b_ref, z_ref, r_ref):
  h = jnp.maximum(_mean_of(aggp_ref[...], cntp_ref[...], r1_ref[...]), 0.0)
  dn = (((1,), (1,)), ((), ()))
  z_ref[...] = lax.dot_general(h, wl_ref[...], dn,
                               preferred_element_type=jnp.float32)
  r_ref[...] = lax.dot_general(h, wr_ref[...], dn,
                               preferred_element_type=jnp.float32) + b_ref[...]


_mid = pl.pallas_call(
    _mid_body,
    grid=(N // BLK,),
    in_specs=[
        pl.BlockSpec((NC, BLK, D), lambda i: (0, i, 0)),
        pl.BlockSpec((NC, BLK, D), lambda i: (0, i, 0)),
        pl.BlockSpec((BLK, D), lambda i: (i, 0)),
        pl.BlockSpec((D, D), lambda i: (0, 0)),
        pl.BlockSpec((D, D), lambda i: (0, 0)),
        pl.BlockSpec((1, D), lambda i: (0, 0)),
    ],
    out_specs=[pl.BlockSpec((BLK, D), lambda i: (i, 0)),
               pl.BlockSpec((BLK, D), lambda i: (i, 0))],
    out_shape=[jax.ShapeDtypeStruct((N, D), jnp.float32)] * 2,
)


def _fin_body(aggp_ref, cntp_ref, r2_ref, o_ref):
  a = _mean_of(aggp_ref[...], cntp_ref[...], r2_ref[...])
  m = jnp.max(a, axis=1, keepdims=True)
  lse = jnp.log(jnp.sum(jnp.exp(a - m), axis=1, keepdims=True)) + m
  o_ref[...] = a - lse


_fin = pl.pallas_call(
    _fin_body,
    grid=(N // BLK,),
    in_specs=[
        pl.BlockSpec((NC, BLK, D), lambda i: (0, i, 0)),
        pl.BlockSpec((NC, BLK, D), lambda i: (0, i, 0)),
        pl.BlockSpec((BLK, D), lambda i: (i, 0)),
    ],
    out_specs=pl.BlockSpec((BLK, D), lambda i: (i, 0)),
    out_shape=jax.ShapeDtypeStruct((N, D), jnp.float32),
)


def kernel(x, edge_index, W1l, b1l, W1r, W2l, b2l, W2r):
  ei = edge_index.astype(jnp.int32)
  pad = jnp.zeros((NW, EPWPX - EPW), jnp.int32)
  src = jnp.concatenate([ei[0].reshape(NW, EPW), pad], axis=1).reshape(-1)
  dst = jnp.concatenate([ei[1].reshape(NW, EPW), pad + N], axis=1).reshape(-1)
  zrow = jnp.zeros((CHUNK, D), jnp.float32)
  onesw = jnp.ones((CHUNK, D), jnp.float32)

  cntp = _cntk(onesw, src, dst, zrow).reshape(NC, NP, D)
  z1, r1 = _lin2(x, W1l, W1r, b1l.reshape(1, D))
  aggp1 = _seg(z1, src, dst, zrow).reshape(NC, NP, D)
  z2, r2 = _mid(aggp1, cntp, r1, W2l, W2r, b2l.reshape(1, D))
  aggp2 = _seg(z2, src, dst, zrow).reshape(NC, NP, D)
  return _fin(aggp2, cntp, r2)


# speedup vs baseline: 1.3879x; 1.3879x over previous
import functools

import jax
import jax.numpy as jnp
from jax import lax
from jax.experimental import pallas as pl
from jax.experimental.pallas import tpu as pltpu
from jax.experimental.pallas import tpu_sc as plsc

N = 10000
D = 128
E = 320000
NC = 2
NS = 16
NW = NC * NS
EPW = E // NW
CHUNK = 128
NCHUNK = 80
EPWP = NCHUNK * CHUNK
NP = 10240
RPS = NP // NS
EPWPX = EPWP + CHUNK
BLK = 1000


def _zero_table(zrow_hbm, rows_v, agg_sh, r0):
  pltpu.sync_copy(zrow_hbm, rows_v)
  for t in range(RPS // CHUNK):
    pltpu.sync_copy(rows_v, agg_sh.at[pl.ds(r0 + t * CHUNK, CHUNK)])


def _copy_out(agg_sh, rows_v, agg_out, cid, r0):
  for t in range(RPS // CHUNK):
    pltpu.sync_copy(agg_sh.at[pl.ds(r0 + t * CHUNK, CHUNK)], rows_v)
    pltpu.sync_copy(rows_v, agg_out.at[pl.ds(cid * NP + r0 + t * CHUNK, CHUNK)])


_seg_mesh = plsc.VectorSubcoreMesh(core_axis_name="c", subcore_axis_name="s")


@functools.partial(
    pl.kernel, mesh=_seg_mesh,
    out_type=jax.ShapeDtypeStruct((NC * NP, D), jnp.float32),
    scratch_types=[
        pltpu.VMEM((CHUNK,), jnp.int32),
        pltpu.VMEM((CHUNK,), jnp.int32),
        pltpu.VMEM((CHUNK,), jnp.int32),
        pltpu.VMEM((CHUNK,), jnp.int32),
        pltpu.VMEM((CHUNK, D), jnp.float32),
        pltpu.VMEM((CHUNK, D), jnp.float32),
        pltpu.VMEM_SHARED((NP, D), jnp.float32),
        pltpu.SemaphoreType.DMA,
        pltpu.SemaphoreType.DMA,
    ])
def _seg(z_hbm, src_hbm, dst_hbm, zrow_hbm, agg_out,
         src_a, dst_a, src_b, dst_b, rows_a, rows_b, agg_sh, sem_a, sem_b):
  cid = lax.axis_index("c")
  sid = lax.axis_index("s")
  wid = sid * NC + cid
  r0 = sid * RPS
  _zero_table(zrow_hbm, rows_a, agg_sh, r0)
  plsc.subcore_barrier()

  ebase = wid * EPWPX
  pltpu.sync_copy(src_hbm.at[pl.ds(ebase, CHUNK)], src_a)
  pltpu.sync_copy(dst_hbm.at[pl.ds(ebase, CHUNK)], dst_a)
  pltpu.async_copy(z_hbm.at[src_a], rows_a, sem_a)

  def step(j, carry):
    c1 = 2 * j + 1
    c2 = 2 * j + 2
    pltpu.sync_copy(src_hbm.at[pl.ds(ebase + c1 * CHUNK, CHUNK)], src_b)
    pltpu.sync_copy(dst_hbm.at[pl.ds(ebase + c1 * CHUNK, CHUNK)], dst_b)
    pltpu.async_copy(z_hbm.at[src_b], rows_b, sem_b)
    pltpu.make_async_copy(z_hbm.at[src_a], rows_a, sem_a).wait()
    pltpu.sync_copy(rows_a, agg_sh.at[dst_a], add=True)
    pltpu.sync_copy(src_hbm.at[pl.ds(ebase + c2 * CHUNK, CHUNK)], src_a)
    pltpu.sync_copy(dst_hbm.at[pl.ds(ebase + c2 * CHUNK, CHUNK)], dst_a)
    pltpu.async_copy(z_hbm.at[src_a], rows_a, sem_a)
    pltpu.make_async_copy(z_hbm.at[src_b], rows_b, sem_b).wait()
    pltpu.sync_copy(rows_b, agg_sh.at[dst_b], add=True)
    return carry

  lax.fori_loop(0, NCHUNK // 2, step, 0)
  pltpu.make_async_copy(z_hbm.at[src_a], rows_a, sem_a).wait()
  plsc.subcore_barrier()
  _copy_out(agg_sh, rows_a, agg_out, cid, r0)


@functools.partial(
    pl.kernel, mesh=_seg_mesh,
    out_type=jax.ShapeDtypeStruct((NC * NP, D), jnp.float32),
    scratch_types=[
        pltpu.VMEM((CHUNK,), jnp.int32),
        pltpu.VMEM((CHUNK,), jnp.int32),
        pltpu.VMEM((CHUNK, D), jnp.float32),
        pltpu.VMEM_SHARED((NP, D), jnp.float32),
        pltpu.SemaphoreType.DMA,
    ])
def _cntk(ones_hbm, src_hbm, dst_hbm, zrow_hbm, cnt_out,
          dst_a, dst_b, rows_v, cnt_sh, sem):
  del src_hbm
  cid = lax.axis_index("c")
  sid = lax.axis_index("s")
  wid = sid * NC + cid
  r0 = sid * RPS
  _zero_table(zrow_hbm, rows_v, cnt_sh, r0)
  pltpu.sync_copy(ones_hbm, rows_v)
  plsc.subcore_barrier()

  ebase = wid * EPWPX
  pltpu.sync_copy(dst_hbm.at[pl.ds(ebase, CHUNK)], dst_a)

  def step(j, carry):
    c1 = 2 * j + 1
    c2 = 2 * j + 2
    pltpu.sync_copy(dst_hbm.at[pl.ds(ebase + c1 * CHUNK, CHUNK)], dst_b)
    pltpu.sync_copy(rows_v, cnt_sh.at[dst_a], add=True)
    pltpu.sync_copy(dst_hbm.at[pl.ds(ebase + c2 * CHUNK, CHUNK)], dst_a)
    pltpu.sync_copy(rows_v, cnt_sh.at[dst_b], add=True)
    return carry

  lax.fori_loop(0, NCHUNK // 2, step, 0)
  plsc.subcore_barrier()
  _copy_out(cnt_sh, rows_v, cnt_out, cid, r0)


def _lin2_body(x_ref, wl_ref, wr_ref, b_ref, z_ref, r_ref):
  x = x_ref[...]
  dn = (((1,), (1,)), ((), ()))
  z_ref[...] = lax.dot_general(x, wl_ref[...], dn,
                               preferred_element_type=jnp.float32)
  r_ref[...] = lax.dot_general(x, wr_ref[...], dn,
                               preferred_element_type=jnp.float32) + b_ref[...]


_lin2 = pl.pallas_call(
    _lin2_body,
    grid=(N // BLK,),
    in_specs=[
        pl.BlockSpec((BLK, D), lambda i: (i, 0)),
        pl.BlockSpec((D, D), lambda i: (0, 0)),
        pl.BlockSpec((D, D), lambda i: (0, 0)),
        pl.BlockSpec((1, D), lambda i: (0, 0)),
    ],
    out_specs=[pl.BlockSpec((BLK, D), lambda i: (i, 0)),
               pl.BlockSpec((BLK, D), lambda i: (i, 0))],
    out_shape=[jax.ShapeDtypeStruct((N, D), jnp.float32)] * 2,
)


def _mean_of(aggp, cntp, res):
  agg = aggp[0] + aggp[1]
  cnt = cntp[0, :, 0:1] + cntp[1, :, 0:1]
  return agg / jnp.clip(cnt, 1.0, None) + res


def _mid_body(aggp_ref, cntp_ref, r1_ref, wl_ref, wr_ref, b_ref, z_ref, r_ref):
  h = jnp.maximum(_mean_of(aggp_ref[...], cntp_ref[...], r1_ref[...]), 0.0)
  dn = (((1,), (1,)), ((), ()))
  z_ref[...] = lax.dot_general(h, wl_ref[...], dn,
                               preferred_element_type=jnp.float32)
  r_ref[...] = lax.dot_general(h, wr_ref[...], dn,
                               preferred_element_type=jnp.float32) + b_ref[...]


_mid = pl.pallas_call(
    _mid_body,
    grid=(N // BLK,),
    in_specs=[
        pl.BlockSpec((NC, BLK, D), lambda i: (0, i, 0)),
        pl.BlockSpec((NC, BLK, D), lambda i: (0, i, 0)),
        pl.BlockSpec((BLK, D), lambda i: (i, 0)),
        pl.BlockSpec((D, D), lambda i: (0, 0)),
        pl.BlockSpec((D, D), lambda i: (0, 0)),
        pl.BlockSpec((1, D), lambda i: (0, 0)),
    ],
    out_specs=[pl.BlockSpec((BLK, D), lambda i: (i, 0)),
               pl.BlockSpec((BLK, D), lambda i: (i, 0))],
    out_shape=[jax.ShapeDtypeStruct((N, D), jnp.float32)] * 2,
)


def _fin_body(aggp_ref, cntp_ref, r2_ref, o_ref):
  a = _mean_of(aggp_ref[...], cntp_ref[...], r2_ref[...])
  m = jnp.max(a, axis=1, keepdims=True)
  lse = jnp.log(jnp.sum(jnp.exp(a - m), axis=1, keepdims=True)) + m
  o_ref[...] = a - lse


_fin = pl.pallas_call(
    _fin_body,
    grid=(N // BLK,),
    in_specs=[
        pl.BlockSpec((NC, BLK, D), lambda i: (0, i, 0)),
        pl.BlockSpec((NC, BLK, D), lambda i: (0, i, 0)),
        pl.BlockSpec((BLK, D), lambda i: (i, 0)),
    ],
    out_specs=pl.BlockSpec((BLK, D), lambda i: (i, 0)),
    out_shape=jax.ShapeDtypeStruct((N, D), jnp.float32),
)


def kernel(x, edge_index, W1l, b1l, W1r, W2l, b2l, W2r):
  ei = edge_index.astype(jnp.int32)
  pad = jnp.zeros((NW, EPWPX - EPW), jnp.int32)
  src = jnp.concatenate([ei[0].reshape(NW, EPW), pad], axis=1).reshape(-1)
  dst = jnp.concatenate([ei[1].reshape(NW, EPW), pad + N], axis=1).reshape(-1)
  zrow = jnp.zeros((CHUNK, D), jnp.float32)
  onesw = jnp.ones((CHUNK, D), jnp.float32)

  cntp = _cntk(onesw, src, dst, zrow).reshape(NC, NP, D)
  z1, r1 = _lin2(x, W1l, W1r, b1l.reshape(1, D))
  aggp1 = _seg(z1, src, dst, zrow).reshape(NC, NP, D)
  z2, r2 = _mid(aggp1, cntp, r1, W2l, W2r, b2l.reshape(1, D))
  aggp2 = _seg(z2, src, dst, zrow).reshape(NC, NP, D)
  return _fin(aggp2, cntp, r2)


# speedup vs baseline: 1.4206x; 1.0236x over previous
import functools

import jax
import jax.numpy as jnp
from jax import lax
from jax.experimental import pallas as pl
from jax.experimental.pallas import tpu as pltpu
from jax.experimental.pallas import tpu_sc as plsc

N = 10000
D = 128
E = 320000
NC = 2
NS = 16
NW = NC * NS
EPW = E // NW
CHUNK = 80
NCHUNK = 128
EPWP = NCHUNK * CHUNK
NP = 10240
RPS = NP // NS
EPWPX = EPWP + CHUNK
BLK = 1000


def _zero_table(zrow_hbm, rows_v, agg_sh, r0):
  pltpu.sync_copy(zrow_hbm, rows_v)
  for t in range(RPS // CHUNK):
    pltpu.sync_copy(rows_v, agg_sh.at[pl.ds(r0 + t * CHUNK, CHUNK)])


def _copy_out(agg_sh, rows_v, agg_out, cid, r0):
  for t in range(RPS // CHUNK):
    pltpu.sync_copy(agg_sh.at[pl.ds(r0 + t * CHUNK, CHUNK)], rows_v)
    pltpu.sync_copy(rows_v, agg_out.at[pl.ds(cid * NP + r0 + t * CHUNK, CHUNK)])


_seg_mesh = plsc.VectorSubcoreMesh(core_axis_name="c", subcore_axis_name="s")


@functools.partial(
    pl.kernel, mesh=_seg_mesh,
    out_type=jax.ShapeDtypeStruct((NC * NP, D), jnp.float32),
    scratch_types=[
        pltpu.VMEM((CHUNK,), jnp.int32),
        pltpu.VMEM((CHUNK,), jnp.int32),
        pltpu.VMEM((CHUNK,), jnp.int32),
        pltpu.VMEM((CHUNK,), jnp.int32),
        pltpu.VMEM((CHUNK, D), jnp.float32),
        pltpu.VMEM((CHUNK, D), jnp.float32),
        pltpu.VMEM_SHARED((NP, D), jnp.float32),
        pltpu.SemaphoreType.DMA,
        pltpu.SemaphoreType.DMA,
    ])
def _seg(z_hbm, src_hbm, dst_hbm, zrow_hbm, agg_out,
         src_a, dst_a, src_b, dst_b, rows_a, rows_b, agg_sh, sem_a, sem_b):
  cid = lax.axis_index("c")
  sid = lax.axis_index("s")
  wid = sid * NC + cid
  r0 = sid * RPS
  _zero_table(zrow_hbm, rows_a, agg_sh, r0)
  plsc.subcore_barrier()

  ebase = wid * EPWPX
  pltpu.sync_copy(src_hbm.at[pl.ds(ebase, CHUNK)], src_a)
  pltpu.sync_copy(dst_hbm.at[pl.ds(ebase, CHUNK)], dst_a)
  pltpu.async_copy(z_hbm.at[src_a], rows_a, sem_a)

  def step(j, carry):
    c1 = 2 * j + 1
    c2 = 2 * j + 2
    pltpu.sync_copy(src_hbm.at[pl.ds(ebase + c1 * CHUNK, CHUNK)], src_b)
    pltpu.sync_copy(dst_hbm.at[pl.ds(ebase + c1 * CHUNK, CHUNK)], dst_b)
    pltpu.async_copy(z_hbm.at[src_b], rows_b, sem_b)
    pltpu.make_async_copy(z_hbm.at[src_a], rows_a, sem_a).wait()
    pltpu.sync_copy(rows_a, agg_sh.at[dst_a], add=True)
    pltpu.sync_copy(src_hbm.at[pl.ds(ebase + c2 * CHUNK, CHUNK)], src_a)
    pltpu.sync_copy(dst_hbm.at[pl.ds(ebase + c2 * CHUNK, CHUNK)], dst_a)
    pltpu.async_copy(z_hbm.at[src_a], rows_a, sem_a)
    pltpu.make_async_copy(z_hbm.at[src_b], rows_b, sem_b).wait()
    pltpu.sync_copy(rows_b, agg_sh.at[dst_b], add=True)
    return carry

  lax.fori_loop(0, NCHUNK // 2, step, 0)
  pltpu.make_async_copy(z_hbm.at[src_a], rows_a, sem_a).wait()
  plsc.subcore_barrier()
  _copy_out(agg_sh, rows_a, agg_out, cid, r0)


@functools.partial(
    pl.kernel, mesh=_seg_mesh,
    out_type=jax.ShapeDtypeStruct((NC * NP, D), jnp.float32),
    scratch_types=[
        pltpu.VMEM((CHUNK,), jnp.int32),
        pltpu.VMEM((CHUNK,), jnp.int32),
        pltpu.VMEM((CHUNK, D), jnp.float32),
        pltpu.VMEM_SHARED((NP, D), jnp.float32),
        pltpu.SemaphoreType.DMA,
    ])
def _cntk(ones_hbm, src_hbm, dst_hbm, zrow_hbm, cnt_out,
          dst_a, dst_b, rows_v, cnt_sh, sem):
  del src_hbm
  cid = lax.axis_index("c")
  sid = lax.axis_index("s")
  wid = sid * NC + cid
  r0 = sid * RPS
  _zero_table(zrow_hbm, rows_v, cnt_sh, r0)
  pltpu.sync_copy(ones_hbm, rows_v)
  plsc.subcore_barrier()

  ebase = wid * EPWPX
  pltpu.sync_copy(dst_hbm.at[pl.ds(ebase, CHUNK)], dst_a)

  def step(j, carry):
    c1 = 2 * j + 1
    c2 = 2 * j + 2
    pltpu.sync_copy(dst_hbm.at[pl.ds(ebase + c1 * CHUNK, CHUNK)], dst_b)
    pltpu.sync_copy(rows_v, cnt_sh.at[dst_a], add=True)
    pltpu.sync_copy(dst_hbm.at[pl.ds(ebase + c2 * CHUNK, CHUNK)], dst_a)
    pltpu.sync_copy(rows_v, cnt_sh.at[dst_b], add=True)
    return carry

  lax.fori_loop(0, NCHUNK // 2, step, 0)
  plsc.subcore_barrier()
  _copy_out(cnt_sh, rows_v, cnt_out, cid, r0)


def _lin2_body(x_ref, wl_ref, wr_ref, b_ref, z_ref, r_ref):
  x = x_ref[...]
  dn = (((1,), (1,)), ((), ()))
  z_ref[...] = lax.dot_general(x, wl_ref[...], dn,
                               preferred_element_type=jnp.float32)
  r_ref[...] = lax.dot_general(x, wr_ref[...], dn,
                               preferred_element_type=jnp.float32) + b_ref[...]


_lin2 = pl.pallas_call(
    _lin2_body,
    grid=(N // BLK,),
    in_specs=[
        pl.BlockSpec((BLK, D), lambda i: (i, 0)),
        pl.BlockSpec((D, D), lambda i: (0, 0)),
        pl.BlockSpec((D, D), lambda i: (0, 0)),
        pl.BlockSpec((1, D), lambda i: (0, 0)),
    ],
    out_specs=[pl.BlockSpec((BLK, D), lambda i: (i, 0)),
               pl.BlockSpec((BLK, D), lambda i: (i, 0))],
    out_shape=[jax.ShapeDtypeStruct((N, D), jnp.float32)] * 2,
)


def _mean_of(aggp, cntp, res):
  agg = aggp[0] + aggp[1]
  cnt = cntp[0, :, 0:1] + cntp[1, :, 0:1]
  return agg / jnp.clip(cnt, 1.0, None) + res


def _mid_body(aggp_ref, cntp_ref, r1_ref, wl_ref, wr_ref, b_ref, z_ref, r_ref):
  h = jnp.maximum(_mean_of(aggp_ref[...], cntp_ref[...], r1_ref[...]), 0.0)
  dn = (((1,), (1,)), ((), ()))
  z_ref[...] = lax.dot_general(h, wl_ref[...], dn,
                               preferred_element_type=jnp.float32)
  r_ref[...] = lax.dot_general(h, wr_ref[...], dn,
                               preferred_element_type=jnp.float32) + b_ref[...]


_mid = pl.pallas_call(
    _mid_body,
    grid=(N // BLK,),
    in_specs=[
        pl.BlockSpec((NC, BLK, D), lambda i: (0, i, 0)),
        pl.BlockSpec((NC, BLK, D), lambda i: (0, i, 0)),
        pl.BlockSpec((BLK, D), lambda i: (i, 0)),
        pl.BlockSpec((D, D), lambda i: (0, 0)),
        pl.BlockSpec((D, D), lambda i: (0, 0)),
        pl.BlockSpec((1, D), lambda i: (0, 0)),
    ],
    out_specs=[pl.BlockSpec((BLK, D), lambda i: (i, 0)),
               pl.BlockSpec((BLK, D), lambda i: (i, 0))],
    out_shape=[jax.ShapeDtypeStruct((N, D), jnp.float32)] * 2,
)


def _fin_body(aggp_ref, cntp_ref, r2_ref, o_ref):
  a = _mean_of(aggp_ref[...], cntp_ref[...], r2_ref[...])
  m = jnp.max(a, axis=1, keepdims=True)
  lse = jnp.log(jnp.sum(jnp.exp(a - m), axis=1, keepdims=True)) + m
  o_ref[...] = a - lse


_fin = pl.pallas_call(
    _fin_body,
    grid=(N // BLK,),
    in_specs=[
        pl.BlockSpec((NC, BLK, D), lambda i: (0, i, 0)),
        pl.BlockSpec((NC, BLK, D), lambda i: (0, i, 0)),
        pl.BlockSpec((BLK, D), lambda i: (i, 0)),
    ],
    out_specs=pl.BlockSpec((BLK, D), lambda i: (i, 0)),
    out_shape=jax.ShapeDtypeStruct((N, D), jnp.float32),
)


def kernel(x, edge_index, W1l, b1l, W1r, W2l, b2l, W2r):
  ei = edge_index.astype(jnp.int32)
  pad = jnp.zeros((NW, EPWPX - EPW), jnp.int32)
  src = jnp.concatenate([ei[0].reshape(NW, EPW), pad], axis=1).reshape(-1)
  dst = jnp.concatenate([ei[1].reshape(NW, EPW), pad + N], axis=1).reshape(-1)
  zrow = jnp.zeros((CHUNK, D), jnp.float32)
  onesw = jnp.ones((CHUNK, D), jnp.float32)

  cntp = _cntk(onesw, src, dst, zrow).reshape(NC, NP, D)
  z1, r1 = _lin2(x, W1l, W1r, b1l.reshape(1, D))
  aggp1 = _seg(z1, src, dst, zrow).reshape(NC, NP, D)
  z2, r2 = _mid(aggp1, cntp, r1, W2l, W2r, b2l.reshape(1, D))
  aggp2 = _seg(z2, src, dst, zrow).reshape(NC, NP, D)
  return _fin(aggp2, cntp, r2)
